# Initial kernel scaffold; baseline (speedup 1.0000x reference)
import jax
import jax.numpy as jnp
from jax.experimental import pallas as pl


def kernel(x, edge_index, edge_attr, W1, b1, W2, b2):
    raise NotImplementedError("write your pallas kernel here")


# speedup vs baseline: 1.5529x; 1.5529x over previous
import jax
import jax.numpy as jnp
from jax import lax
from jax.experimental import pallas as pl
from jax.experimental.pallas import tpu as pltpu
from jax.experimental.pallas import tpu_sc as plsc

N_NODES = 10000
N_EDGES = 320000
H = 128
HH = H // 2
ACC_W = 80

NC = 2
NS = 16
CHUNK = 128
N_CHUNKS = N_EDGES // CHUNK
N_PAD = 10240
ROWS_PER_TILE = N_PAD // NS


def _xw_body(x_ref, wa_ref, wb_ref, xa_ref, xb_ref):
    xv = x_ref[...]
    xa_ref[...] = jnp.dot(xv, wa_ref[0], preferred_element_type=jnp.float32)
    xb_ref[...] = jnp.dot(xv, wb_ref[0], preferred_element_type=jnp.float32)


def _node_tables(x, w1a, w1b):
    bm = 1000
    nb = N_NODES // bm
    return pl.pallas_call(
        _xw_body,
        grid=(NC, nb),
        in_specs=[
            pl.BlockSpec((bm, H), lambda h, i: (i, 0)),
            pl.BlockSpec((1, H, HH), lambda h, i: (h, 0, 0)),
            pl.BlockSpec((1, H, HH), lambda h, i: (h, 0, 0)),
        ],
        out_specs=[
            pl.BlockSpec((bm, HH), lambda h, i: (h * nb + i, 0)),
            pl.BlockSpec((bm, HH), lambda h, i: (h * nb + i, 0)),
        ],
        out_shape=[
            jax.ShapeDtypeStruct((NC * N_NODES, HH), jnp.float32),
            jax.ShapeDtypeStruct((NC * N_NODES, HH), jnp.float32),
        ],
    )(x, w1a, w1b)


def _ec_body(ea_ref, wc_ref, b1_ref, out_ref):
    m = jnp.dot(ea_ref[...], wc_ref[0],
                preferred_element_type=jnp.float32) + b1_ref[0]
    bm = m.shape[0]
    col = lax.broadcasted_iota(jnp.int32, (bm, ACC_W - HH), 1)
    tail = jnp.where(col == 0, 1.0, 0.0).astype(jnp.float32)
    out_ref[...] = jnp.concatenate([m, tail], axis=1)


def _edge_table(edge_attr, w1c, b1):
    bm = 2000
    nb = N_EDGES // bm
    return pl.pallas_call(
        _ec_body,
        grid=(NC, nb),
        in_specs=[
            pl.BlockSpec((bm, 16), lambda h, i: (i, 0)),
            pl.BlockSpec((1, 16, HH), lambda h, i: (h, 0, 0)),
            pl.BlockSpec((1, 1, HH), lambda h, i: (h, 0, 0)),
        ],
        out_specs=pl.BlockSpec((bm, ACC_W), lambda h, i: (h * nb + i, 0)),
        out_shape=jax.ShapeDtypeStruct((NC * N_EDGES, ACC_W), jnp.float32),
    )(edge_attr, w1c, b1)


def _fin_body(x_ref, acc_ref, w2a_ref, w2b_ref, b2_ref, out_ref):
    a0 = acc_ref[0]
    a1 = acc_ref[1]
    s = jnp.concatenate([a0[:, :HH], a1[:, :HH]], axis=1)
    cnt = a0[:, HH:HH + 1]
    mean = s / jnp.maximum(cnt, 1.0)
    r = jnp.dot(x_ref[...], w2a_ref[...], preferred_element_type=jnp.float32)
    r = r + jnp.dot(mean, w2b_ref[...], preferred_element_type=jnp.float32)
    out_ref[...] = jnp.maximum(r + b2_ref[...], 0.0)


def _finalize(x, acc, w2a, w2b, b2):
    bm = 1000
    return pl.pallas_call(
        _fin_body,
        grid=(N_NODES // bm,),
        in_specs=[
            pl.BlockSpec((bm, H), lambda i: (i, 0)),
            pl.BlockSpec((NC, bm, ACC_W), lambda i: (0, i, 0)),
            pl.BlockSpec((H, H), lambda i: (0, 0)),
            pl.BlockSpec((H, H), lambda i: (0, 0)),
            pl.BlockSpec((1, H), lambda i: (0, 0)),
        ],
        out_specs=pl.BlockSpec((bm, H), lambda i: (i, 0)),
        out_shape=jax.ShapeDtypeStruct((N_NODES, H), jnp.float32),
    )(x, acc, w2a, w2b, b2)


def _sc_body(xa_hbm, xb_hbm, ei_hbm, ec_hbm, zeros_hbm, out_hbm,
             ridx, cidx, gidx, gx, gy, hbuf, acc, sem1, sem2):
    c = lax.axis_index("c")
    s = lax.axis_index("s")

    pltpu.sync_copy(zeros_hbm, acc.at[pl.ds(s * ROWS_PER_TILE, ROWS_PER_TILE)])
    plsc.subcore_barrier()

    n_chunks = (N_CHUNKS // NS) + jnp.where(s < N_CHUNKS % NS, 1, 0)
    cbase = jnp.full((16,), c * N_NODES, jnp.int32)

    def chunk_body(k, carry):
        g = s + k * NS
        off = g * CHUNK
        pltpu.sync_copy(ei_hbm.at[0, pl.ds(off, CHUNK)], ridx)
        pltpu.sync_copy(ei_hbm.at[1, pl.ds(off, CHUNK)], cidx)
        for j in range(CHUNK // 16):
            sl = pl.ds(j * 16, 16)
            ridx[sl] = ridx[sl] + cbase
            gidx[sl] = cidx[sl] + cbase
        cp1 = pltpu.async_copy(xa_hbm.at[ridx], gx, sem1)
        cp2 = pltpu.async_copy(xb_hbm.at[gidx], gy, sem2)
        pltpu.sync_copy(ec_hbm.at[pl.ds(c * N_EDGES + off, CHUNK), :], hbuf)
        cp1.wait()
        cp2.wait()

        def row_body(i, carry2):
            for j in range(HH // 16):
                sl = pl.ds(j * 16, 16)
                hbuf[i, sl] = jnp.maximum(gx[i, sl] + gy[i, sl] + hbuf[i, sl],
                                          0.0)
            return carry2

        lax.fori_loop(0, CHUNK, row_body, 0)
        pltpu.sync_copy(hbuf, acc.at[cidx], add=True)
        return carry

    lax.fori_loop(0, n_chunks, chunk_body, 0)
    plsc.subcore_barrier()
    pltpu.sync_copy(acc.at[pl.ds(s * ROWS_PER_TILE, ROWS_PER_TILE)],
                    out_hbm.at[c, pl.ds(s * ROWS_PER_TILE, ROWS_PER_TILE), :])


def _sc_aggregate(xa, xb, edge_index, ec, zeros):
    mesh = plsc.VectorSubcoreMesh(core_axis_name="c", subcore_axis_name="s")
    run = pl.kernel(
        _sc_body,
        mesh=mesh,
        compiler_params=pltpu.CompilerParams(use_tc_tiling_on_sc=False),
        out_type=jax.ShapeDtypeStruct((NC, N_PAD, ACC_W), jnp.float32),
        scratch_types=[
            pltpu.VMEM((CHUNK,), jnp.int32),
            pltpu.VMEM((CHUNK,), jnp.int32),
            pltpu.VMEM((CHUNK,), jnp.int32),
            pltpu.VMEM((CHUNK, HH), jnp.float32),
            pltpu.VMEM((CHUNK, HH), jnp.float32),
            pltpu.VMEM((CHUNK, ACC_W), jnp.float32),
            pltpu.VMEM_SHARED((N_PAD, ACC_W), jnp.float32),
            pltpu.SemaphoreType.DMA,
            pltpu.SemaphoreType.DMA,
        ],
    )
    return run(xa, xb, edge_index, ec, zeros)


def kernel(x, edge_index, edge_attr, W1, b1, W2, b2):
    w1a = W1[:H].reshape(H, NC, HH).transpose(1, 0, 2)
    w1b = W1[H:2 * H].reshape(H, NC, HH).transpose(1, 0, 2)
    w1c = W1[2 * H:].reshape(16, NC, HH).transpose(1, 0, 2)
    w2a = W2[:H]
    w2b = W2[H:]
    b1r = b1.reshape(1, NC, HH).transpose(1, 0, 2)
    b2r = b2.reshape(1, H)

    xa, xb = _node_tables(x, w1a, w1b)
    ec = _edge_table(edge_attr, w1c, b1r)
    zeros = jnp.zeros((ROWS_PER_TILE, ACC_W), jnp.float32)
    acc = _sc_aggregate(xa, xb, edge_index, ec, zeros)
    return _finalize(x, acc, w2a, w2b, b2r)


# speedup vs baseline: 1.8539x; 1.1938x over previous
import jax
import jax.numpy as jnp
from jax import lax
from jax.experimental import pallas as pl
from jax.experimental.pallas import tpu as pltpu
from jax.experimental.pallas import tpu_sc as plsc

N_NODES = 10000
N_EDGES = 320000
H = 128
HH = H // 2
ACC_W = 80

NC = 2
NS = 16
CHUNK = 128
N_CHUNKS = N_EDGES // CHUNK
N_PAD = 10240
ROWS_PER_TILE = N_PAD // NS


def _xw_body(x_ref, wa_ref, wb_ref, xa_ref, xb_ref):
    xv = x_ref[...]
    xa_ref[...] = jnp.dot(xv, wa_ref[0], preferred_element_type=jnp.float32)
    xb_ref[...] = jnp.dot(xv, wb_ref[0], preferred_element_type=jnp.float32)


def _node_tables(x, w1a, w1b):
    bm = 1000
    nb = N_NODES // bm
    return pl.pallas_call(
        _xw_body,
        grid=(NC, nb),
        in_specs=[
            pl.BlockSpec((bm, H), lambda h, i: (i, 0)),
            pl.BlockSpec((1, H, HH), lambda h, i: (h, 0, 0)),
            pl.BlockSpec((1, H, HH), lambda h, i: (h, 0, 0)),
        ],
        out_specs=[
            pl.BlockSpec((bm, HH), lambda h, i: (h * nb + i, 0)),
            pl.BlockSpec((bm, HH), lambda h, i: (h * nb + i, 0)),
        ],
        out_shape=[
            jax.ShapeDtypeStruct((NC * N_NODES, HH), jnp.float32),
            jax.ShapeDtypeStruct((NC * N_NODES, HH), jnp.float32),
        ],
    )(x, w1a, w1b)


def _ec_body(ea_ref, wc_ref, b1_ref, out_ref):
    m = jnp.dot(ea_ref[...], wc_ref[0],
                preferred_element_type=jnp.float32) + b1_ref[0]
    bm = m.shape[0]
    col = lax.broadcasted_iota(jnp.int32, (bm, ACC_W - HH), 1)
    tail = jnp.where(col == 0, 1.0, 0.0).astype(jnp.float32)
    out_ref[...] = jnp.concatenate([m, tail], axis=1)


def _edge_table(edge_attr, w1c, b1):
    bm = 2000
    nb = N_EDGES // bm
    return pl.pallas_call(
        _ec_body,
        grid=(NC, nb),
        in_specs=[
            pl.BlockSpec((bm, 16), lambda h, i: (i, 0)),
            pl.BlockSpec((1, 16, HH), lambda h, i: (h, 0, 0)),
            pl.BlockSpec((1, 1, HH), lambda h, i: (h, 0, 0)),
        ],
        out_specs=pl.BlockSpec((bm, ACC_W), lambda h, i: (h * nb + i, 0)),
        out_shape=jax.ShapeDtypeStruct((NC * N_EDGES, ACC_W), jnp.float32),
    )(edge_attr, w1c, b1)


def _fin_body(x_ref, acc_ref, w2a_ref, w2b_ref, b2_ref, out_ref):
    a0 = acc_ref[0]
    a1 = acc_ref[1]
    s = jnp.concatenate([a0[:, :HH], a1[:, :HH]], axis=1)
    cnt = a0[:, HH:HH + 1]
    mean = s / jnp.maximum(cnt, 1.0)
    r = jnp.dot(x_ref[...], w2a_ref[...], preferred_element_type=jnp.float32)
    r = r + jnp.dot(mean, w2b_ref[...], preferred_element_type=jnp.float32)
    out_ref[...] = jnp.maximum(r + b2_ref[...], 0.0)


def _finalize(x, acc, w2a, w2b, b2):
    bm = 1000
    return pl.pallas_call(
        _fin_body,
        grid=(N_NODES // bm,),
        in_specs=[
            pl.BlockSpec((bm, H), lambda i: (i, 0)),
            pl.BlockSpec((NC, bm, ACC_W), lambda i: (0, i, 0)),
            pl.BlockSpec((H, H), lambda i: (0, 0)),
            pl.BlockSpec((H, H), lambda i: (0, 0)),
            pl.BlockSpec((1, H), lambda i: (0, 0)),
        ],
        out_specs=pl.BlockSpec((bm, H), lambda i: (i, 0)),
        out_shape=jax.ShapeDtypeStruct((N_NODES, H), jnp.float32),
    )(x, acc, w2a, w2b, b2)


def _sc_body(xa_hbm, xb_hbm, ei_hbm, ec_hbm, zeros_hbm, out_hbm,
             ridx0, cidx0, gidx0, gx0, gy0, h0,
             ridx1, cidx1, gidx1, gx1, gy1, h1,
             acc,
             isem0, gsem0, esem0, ssem0,
             isem1, gsem1, esem1, ssem1):
    c = lax.axis_index("c")
    s = lax.axis_index("s")

    pltpu.sync_copy(zeros_hbm, acc.at[pl.ds(s * ROWS_PER_TILE, ROWS_PER_TILE)])
    plsc.subcore_barrier()

    n = (N_CHUNKS // NS) + jnp.where(s < N_CHUNKS % NS, 1, 0)
    start = s * (N_CHUNKS // NS) + jnp.minimum(s, N_CHUNKS % NS)
    cbase = jnp.full((16,), c * N_NODES, jnp.int32)

    sets = (
        (ridx0, cidx0, gidx0, gx0, gy0, h0, isem0, gsem0, esem0, ssem0),
        (ridx1, cidx1, gidx1, gx1, gy1, h1, isem1, gsem1, esem1, ssem1),
    )

    def prefetch(k, S, drain):
        ridx, cidx, gidx, gx, gy, h, isem, gsem, esem, ssem = S

        @pl.when(k < n)
        def _():
            if drain:
                @pl.when(k >= 2)
                def _():
                    pltpu.make_async_copy(h, acc.at[cidx], ssem).wait()
            off = (start + k) * CHUNK
            d1 = pltpu.async_copy(ei_hbm.at[0, pl.ds(off, CHUNK)], ridx, isem)
            d2 = pltpu.async_copy(ei_hbm.at[1, pl.ds(off, CHUNK)], cidx, isem)
            d1.wait()
            d2.wait()
            for j in range(CHUNK // 16):
                sl = pl.ds(j * 16, 16)
                ridx[sl] = ridx[sl] + cbase
                gidx[sl] = cidx[sl] + cbase
            pltpu.async_copy(xa_hbm.at[ridx], gx, gsem)
            pltpu.async_copy(xb_hbm.at[gidx], gy, gsem)
            pltpu.async_copy(ec_hbm.at[pl.ds(c * N_EDGES + off, CHUNK), :],
                             h, esem)

    def process(k, S):
        ridx, cidx, gidx, gx, gy, h, isem, gsem, esem, ssem = S

        @pl.when(k < n)
        def _():
            pltpu.make_async_copy(xa_hbm.at[ridx], gx, gsem).wait()
            pltpu.make_async_copy(xb_hbm.at[gidx], gy, gsem).wait()
            pltpu.make_async_copy(ec_hbm.at[pl.ds(0, CHUNK), :], h,
                                  esem).wait()

            def row_body(i, carry2):
                for j in range(HH // 16):
                    sl = pl.ds(j * 16, 16)
                    h[i, sl] = jnp.maximum(gx[i, sl] + gy[i, sl] + h[i, sl],
                                           0.0)
                return carry2

            lax.fori_loop(0, CHUNK, row_body, 0)
            pltpu.async_copy(h, acc.at[cidx], ssem, add=True)

    prefetch(jnp.int32(0), sets[0], drain=False)

    def pair_body(j, carry):
        k0 = 2 * j
        prefetch(k0 + 1, sets[1], drain=True)
        process(k0, sets[0])
        prefetch(k0 + 2, sets[0], drain=True)
        process(k0 + 1, sets[1])
        return carry

    lax.fori_loop(0, (n + 1) // 2, pair_body, 0)

    pltpu.make_async_copy(h0, acc.at[cidx0], ssem0).wait()
    pltpu.make_async_copy(h1, acc.at[cidx1], ssem1).wait()
    plsc.subcore_barrier()
    pltpu.sync_copy(acc.at[pl.ds(s * ROWS_PER_TILE, ROWS_PER_TILE)],
                    out_hbm.at[c, pl.ds(s * ROWS_PER_TILE, ROWS_PER_TILE), :])


def _sc_aggregate(xa, xb, edge_index, ec, zeros):
    mesh = plsc.VectorSubcoreMesh(core_axis_name="c", subcore_axis_name="s")
    buf = [
        pltpu.VMEM((CHUNK,), jnp.int32),
        pltpu.VMEM((CHUNK,), jnp.int32),
        pltpu.VMEM((CHUNK,), jnp.int32),
        pltpu.VMEM((CHUNK, HH), jnp.float32),
        pltpu.VMEM((CHUNK, HH), jnp.float32),
        pltpu.VMEM((CHUNK, ACC_W), jnp.float32),
    ]
    run = pl.kernel(
        _sc_body,
        mesh=mesh,
        compiler_params=pltpu.CompilerParams(use_tc_tiling_on_sc=False),
        out_type=jax.ShapeDtypeStruct((NC, N_PAD, ACC_W), jnp.float32),
        scratch_types=buf + buf + [
            pltpu.VMEM_SHARED((N_PAD, ACC_W), jnp.float32),
        ] + [pltpu.SemaphoreType.DMA] * 8,
    )
    return run(xa, xb, edge_index, ec, zeros)


def kernel(x, edge_index, edge_attr, W1, b1, W2, b2):
    w1a = W1[:H].reshape(H, NC, HH).transpose(1, 0, 2)
    w1b = W1[H:2 * H].reshape(H, NC, HH).transpose(1, 0, 2)
    w1c = W1[2 * H:].reshape(16, NC, HH).transpose(1, 0, 2)
    w2a = W2[:H]
    w2b = W2[H:]
    b1r = b1.reshape(1, NC, HH).transpose(1, 0, 2)
    b2r = b2.reshape(1, H)

    xa, xb = _node_tables(x, w1a, w1b)
    ec = _edge_table(edge_attr, w1c, b1r)
    zeros = jnp.zeros((ROWS_PER_TILE, ACC_W), jnp.float32)
    acc = _sc_aggregate(xa, xb, edge_index, ec, zeros)
    return _finalize(x, acc, w2a, w2b, b2r)


# speedup vs baseline: 2.9836x; 1.6093x over previous
import jax
import jax.numpy as jnp
from jax import lax
from jax.experimental import pallas as pl
from jax.experimental.pallas import tpu as pltpu
from jax.experimental.pallas import tpu_sc as plsc

N_NODES = 10000
N_EDGES = 320000
H = 128
HH = H // 2
ACC_W = 80

NC = 2
NS = 16
CHUNK = 128
N_CHUNKS = N_EDGES // CHUNK
N_PAD = 10240
ROWS_PER_TILE = N_PAD // NS


def _xw_body(x_ref, wa_ref, wb_ref, xa_ref, xb_ref):
    xv = x_ref[...]
    xa_ref[...] = jnp.dot(xv, wa_ref[0], preferred_element_type=jnp.float32)
    xb_ref[...] = jnp.dot(xv, wb_ref[0], preferred_element_type=jnp.float32)


def _node_tables(x, w1a, w1b):
    bm = 1000
    nb = N_NODES // bm
    return pl.pallas_call(
        _xw_body,
        grid=(NC, nb),
        in_specs=[
            pl.BlockSpec((bm, H), lambda h, i: (i, 0)),
            pl.BlockSpec((1, H, HH), lambda h, i: (h, 0, 0)),
            pl.BlockSpec((1, H, HH), lambda h, i: (h, 0, 0)),
        ],
        out_specs=[
            pl.BlockSpec((bm, HH), lambda h, i: (h * nb + i, 0)),
            pl.BlockSpec((bm, HH), lambda h, i: (h * nb + i, 0)),
        ],
        out_shape=[
            jax.ShapeDtypeStruct((NC * N_NODES, HH), jnp.float32),
            jax.ShapeDtypeStruct((NC * N_NODES, HH), jnp.float32),
        ],
    )(x, w1a, w1b)


def _ec_body(ea_ref, wc_ref, b1_ref, out_ref):
    out_ref[...] = jnp.dot(ea_ref[...], wc_ref[...],
                           preferred_element_type=jnp.float32) + b1_ref[...]


def _edge_table(edge_attr, w1c, b1):
    bm = 4000
    nb = N_EDGES // bm
    return pl.pallas_call(
        _ec_body,
        grid=(nb,),
        in_specs=[
            pl.BlockSpec((bm, 16), lambda i: (i, 0)),
            pl.BlockSpec((16, H), lambda i: (0, 0)),
            pl.BlockSpec((1, H), lambda i: (0, 0)),
        ],
        out_specs=pl.BlockSpec((bm, H), lambda i: (i, 0)),
        out_shape=jax.ShapeDtypeStruct((N_EDGES, H), jnp.float32),
    )(edge_attr, w1c, b1)


def _fin_body(x_ref, acc_ref, w2a_ref, w2b_ref, b2_ref, out_ref):
    a0 = acc_ref[0]
    a1 = acc_ref[1]
    s = jnp.concatenate([a0[:, :HH], a1[:, :HH]], axis=1)
    cnt = a0[:, HH:HH + 1]
    mean = s / jnp.maximum(cnt, 1.0)
    r = jnp.dot(x_ref[...], w2a_ref[...], preferred_element_type=jnp.float32)
    r = r + jnp.dot(mean, w2b_ref[...], preferred_element_type=jnp.float32)
    out_ref[...] = jnp.maximum(r + b2_ref[...], 0.0)


def _finalize(x, acc, w2a, w2b, b2):
    bm = 1000
    return pl.pallas_call(
        _fin_body,
        grid=(N_NODES // bm,),
        in_specs=[
            pl.BlockSpec((bm, H), lambda i: (i, 0)),
            pl.BlockSpec((NC, bm, ACC_W), lambda i: (0, i, 0)),
            pl.BlockSpec((H, H), lambda i: (0, 0)),
            pl.BlockSpec((H, H), lambda i: (0, 0)),
            pl.BlockSpec((1, H), lambda i: (0, 0)),
        ],
        out_specs=pl.BlockSpec((bm, H), lambda i: (i, 0)),
        out_shape=jax.ShapeDtypeStruct((N_NODES, H), jnp.float32),
    )(x, acc, w2a, w2b, b2)


def _sc_body(xa_hbm, xb_hbm, ei_hbm, ec_hbm, zeros_hbm, out_hbm,
             ridx0, cidx0, gidx0, gx0, gy0, eb0, h0,
             ridx1, cidx1, gidx1, gx1, gy1, eb1, h1,
             acc,
             isem0, gsem0, esem0, ssem0,
             isem1, gsem1, esem1, ssem1):
    c = lax.axis_index("c")
    s = lax.axis_index("s")

    pltpu.sync_copy(zeros_hbm, acc.at[pl.ds(s * ROWS_PER_TILE, ROWS_PER_TILE)])

    one16 = jnp.where(lax.iota(jnp.int32, 16) == 0, 1.0, 0.0)

    def preset_body(i, carry):
        h0[i, pl.ds(HH, 16)] = one16
        h1[i, pl.ds(HH, 16)] = one16
        return carry

    lax.fori_loop(0, CHUNK, preset_body, 0)
    plsc.subcore_barrier()

    n = (N_CHUNKS // NS) + jnp.where(s < N_CHUNKS % NS, 1, 0)
    start = s * (N_CHUNKS // NS) + jnp.minimum(s, N_CHUNKS % NS)
    cbase = jnp.full((16,), c * N_NODES, jnp.int32)

    sets = (
        (ridx0, cidx0, gidx0, gx0, gy0, eb0, h0, isem0, gsem0, esem0, ssem0),
        (ridx1, cidx1, gidx1, gx1, gy1, eb1, h1, isem1, gsem1, esem1, ssem1),
    )

    def prefetch(k, S, drain):
        ridx, cidx, gidx, gx, gy, eb, h, isem, gsem, esem, ssem = S

        @pl.when(k < n)
        def _():
            if drain:
                @pl.when(k >= 2)
                def _():
                    pltpu.make_async_copy(h, acc.at[cidx], ssem).wait()
            off = (start + k) * CHUNK
            d1 = pltpu.async_copy(ei_hbm.at[0, pl.ds(off, CHUNK)], ridx, isem)
            d2 = pltpu.async_copy(ei_hbm.at[1, pl.ds(off, CHUNK)], cidx, isem)
            d1.wait()
            d2.wait()
            for j in range(CHUNK // 16):
                sl = pl.ds(j * 16, 16)
                ridx[sl] = ridx[sl] + cbase
                gidx[sl] = cidx[sl] + cbase
            pltpu.async_copy(xa_hbm.at[ridx], gx, gsem)
            pltpu.async_copy(xb_hbm.at[gidx], gy, gsem)
            pltpu.async_copy(ec_hbm.at[pl.ds(off, CHUNK), pl.ds(c * HH, HH)],
                             eb, esem)

    def process(k, S):
        ridx, cidx, gidx, gx, gy, eb, h, isem, gsem, esem, ssem = S

        @pl.when(k < n)
        def _():
            pltpu.make_async_copy(xa_hbm.at[ridx], gx, gsem).wait()
            pltpu.make_async_copy(xb_hbm.at[gidx], gy, gsem).wait()
            pltpu.make_async_copy(ec_hbm.at[pl.ds(0, CHUNK), pl.ds(0, HH)],
                                  eb, esem).wait()

            def row_body(i, carry2):
                for j in range(HH // 16):
                    sl = pl.ds(j * 16, 16)
                    h[i, sl] = jnp.maximum(gx[i, sl] + gy[i, sl] + eb[i, sl],
                                           0.0)
                return carry2

            lax.fori_loop(0, CHUNK, row_body, 0)
            pltpu.async_copy(h, acc.at[cidx], ssem, add=True)

    prefetch(jnp.int32(0), sets[0], drain=False)

    def pair_body(j, carry):
        k0 = 2 * j
        prefetch(k0 + 1, sets[1], drain=True)
        process(k0, sets[0])
        prefetch(k0 + 2, sets[0], drain=True)
        process(k0 + 1, sets[1])
        return carry

    lax.fori_loop(0, (n + 1) // 2, pair_body, 0)

    pltpu.make_async_copy(h0, acc.at[cidx0], ssem0).wait()
    pltpu.make_async_copy(h1, acc.at[cidx1], ssem1).wait()
    plsc.subcore_barrier()
    pltpu.sync_copy(acc.at[pl.ds(s * ROWS_PER_TILE, ROWS_PER_TILE)],
                    out_hbm.at[c, pl.ds(s * ROWS_PER_TILE, ROWS_PER_TILE), :])


def _sc_aggregate(xa, xb, edge_index, ec, zeros):
    mesh = plsc.VectorSubcoreMesh(core_axis_name="c", subcore_axis_name="s")
    buf = [
        pltpu.VMEM((CHUNK,), jnp.int32),
        pltpu.VMEM((CHUNK,), jnp.int32),
        pltpu.VMEM((CHUNK,), jnp.int32),
        pltpu.VMEM((CHUNK, HH), jnp.float32),
        pltpu.VMEM((CHUNK, HH), jnp.float32),
        pltpu.VMEM((CHUNK, HH), jnp.float32),
        pltpu.VMEM((CHUNK, ACC_W), jnp.float32),
    ]
    run = pl.kernel(
        _sc_body,
        mesh=mesh,
        compiler_params=pltpu.CompilerParams(use_tc_tiling_on_sc=False),
        out_type=jax.ShapeDtypeStruct((NC, N_PAD, ACC_W), jnp.float32),
        scratch_types=buf + buf + [
            pltpu.VMEM_SHARED((N_PAD, ACC_W), jnp.float32),
        ] + [pltpu.SemaphoreType.DMA] * 8,
    )
    return run(xa, xb, edge_index, ec, zeros)


def kernel(x, edge_index, edge_attr, W1, b1, W2, b2):
    w1a = W1[:H].reshape(H, NC, HH).transpose(1, 0, 2)
    w1b = W1[H:2 * H].reshape(H, NC, HH).transpose(1, 0, 2)
    w1c = W1[2 * H:]
    w2a = W2[:H]
    w2b = W2[H:]
    b1r = b1.reshape(1, H)
    b2r = b2.reshape(1, H)

    xa, xb = _node_tables(x, w1a, w1b)
    ec = _edge_table(edge_attr, w1c, b1r)
    zeros = jnp.zeros((ROWS_PER_TILE, ACC_W), jnp.float32)
    acc = _sc_aggregate(xa, xb, edge_index, ec, zeros)
    return _finalize(x, acc, w2a, w2b, b2r)


# speedup vs baseline: 5.1353x; 1.7212x over previous
import jax
import jax.numpy as jnp
from jax import lax
from jax.experimental import pallas as pl
from jax.experimental.pallas import tpu as pltpu
from jax.experimental.pallas import tpu_sc as plsc

N_NODES = 10000
N_EDGES = 320000
H = 128
HH = H // 2
ACC_W = 80

NC = 2
NS = 16
CHUNK = 128
N_CHUNKS = N_EDGES // CHUNK
N_PAD = 10240
ROWS_PER_TILE = N_PAD // NS


def _xw_body(x_ref, wa_ref, wb_ref, xa_ref, xb_ref):
    xv = x_ref[...]
    xa_ref[...] = jnp.dot(xv, wa_ref[...], preferred_element_type=jnp.float32)
    xb_ref[...] = jnp.dot(xv, wb_ref[...], preferred_element_type=jnp.float32)


def _node_tables(x, w1a, w1b):
    bm = 1000
    nb = N_NODES // bm
    return pl.pallas_call(
        _xw_body,
        grid=(nb,),
        in_specs=[
            pl.BlockSpec((bm, H), lambda i: (i, 0)),
            pl.BlockSpec((H, H), lambda i: (0, 0)),
            pl.BlockSpec((H, H), lambda i: (0, 0)),
        ],
        out_specs=[
            pl.BlockSpec((bm, H), lambda i: (i, 0)),
            pl.BlockSpec((bm, H), lambda i: (i, 0)),
        ],
        out_shape=[
            jax.ShapeDtypeStruct((N_NODES, H), jnp.float32),
            jax.ShapeDtypeStruct((N_NODES, H), jnp.float32),
        ],
    )(x, w1a, w1b)


def _ec_body(ea_ref, wc_ref, b1_ref, out_ref):
    out_ref[...] = jnp.dot(ea_ref[...], wc_ref[...],
                           preferred_element_type=jnp.float32) + b1_ref[...]


def _edge_table(edge_attr, w1c, b1):
    bm = 4000
    nb = N_EDGES // bm
    return pl.pallas_call(
        _ec_body,
        grid=(nb,),
        in_specs=[
            pl.BlockSpec((bm, 16), lambda i: (i, 0)),
            pl.BlockSpec((16, H), lambda i: (0, 0)),
            pl.BlockSpec((1, H), lambda i: (0, 0)),
        ],
        out_specs=pl.BlockSpec((bm, H), lambda i: (i, 0)),
        out_shape=jax.ShapeDtypeStruct((N_EDGES, H), jnp.float32),
    )(edge_attr, w1c, b1)


def _fin_body(x_ref, msg_ref, cnt_ref, w2a_ref, w2b_ref, b2_ref, out_ref):
    cnt = cnt_ref[0]
    mean = msg_ref[...] / jnp.maximum(cnt, 1.0)
    r = jnp.dot(x_ref[...], w2a_ref[...], preferred_element_type=jnp.float32)
    r = r + jnp.dot(mean, w2b_ref[...], preferred_element_type=jnp.float32)
    out_ref[...] = jnp.maximum(r + b2_ref[...], 0.0)


def _finalize(x, msg, cnt, w2a, w2b, b2):
    bm = 1000
    return pl.pallas_call(
        _fin_body,
        grid=(N_NODES // bm,),
        in_specs=[
            pl.BlockSpec((bm, H), lambda i: (i, 0)),
            pl.BlockSpec((bm, H), lambda i: (i, 0)),
            pl.BlockSpec((1, bm, 1), lambda i: (0, i, 0)),
            pl.BlockSpec((H, H), lambda i: (0, 0)),
            pl.BlockSpec((H, H), lambda i: (0, 0)),
            pl.BlockSpec((1, H), lambda i: (0, 0)),
        ],
        out_specs=pl.BlockSpec((bm, H), lambda i: (i, 0)),
        out_shape=jax.ShapeDtypeStruct((N_NODES, H), jnp.float32),
    )(x, msg, cnt, w2a, w2b, b2)


def _sc_body(xa_hbm, xb_hbm, row_hbm, col_hbm, ec_hbm, zeros_hbm,
             xas_hbm, xbs_hbm, msg_hbm, cnt_hbm,
             gx0, gy0, eb0, h0,
             gx1, gy1, eb1, h1,
             ridxA, cidxA, gidxA, ridxB, cidxB, gidxB,
             acc, cntb,
             isem0, gsem0, esem0, ssem0,
             isem1, gsem1, esem1, ssem1):
    c = lax.axis_index("c")
    s = lax.axis_index("s")

    pltpu.sync_copy(zeros_hbm, acc.at[pl.ds(s * ROWS_PER_TILE, ROWS_PER_TILE)])

    nt_blk = 125

    def restack(i, carry):
        r0 = s * (N_NODES // NS) + i * nt_blk
        csl = pl.ds(c * HH, HH)
        pltpu.sync_copy(xa_hbm.at[pl.ds(r0, nt_blk), csl],
                        gx0.at[pl.ds(0, nt_blk)])
        pltpu.sync_copy(gx0.at[pl.ds(0, nt_blk)],
                        xas_hbm.at[pl.ds(c * N_NODES + r0, nt_blk), :])
        pltpu.sync_copy(xb_hbm.at[pl.ds(r0, nt_blk), csl],
                        gy0.at[pl.ds(0, nt_blk)])
        pltpu.sync_copy(gy0.at[pl.ds(0, nt_blk)],
                        xbs_hbm.at[pl.ds(c * N_NODES + r0, nt_blk), :])
        return carry

    lax.fori_loop(0, (N_NODES // NS) // nt_blk, restack, 0)

    one16 = jnp.where(lax.iota(jnp.int32, 16) == 0, 1.0, 0.0)

    def preset_body(i, carry):
        h0[i, pl.ds(HH, 16)] = one16
        h1[i, pl.ds(HH, 16)] = one16
        return carry

    lax.fori_loop(0, CHUNK, preset_body, 0)
    plsc.subcore_barrier()

    n = (N_CHUNKS // NS) + jnp.where(s < N_CHUNKS % NS, 1, 0)
    start = s * (N_CHUNKS // NS) + jnp.minimum(s, N_CHUNKS % NS)
    cbase = jnp.full((16,), c * N_NODES, jnp.int32)

    sets = (
        (gx0, gy0, eb0, h0, gsem0, esem0, ssem0),
        (gx1, gy1, eb1, h1, gsem1, esem1, ssem1),
    )
    blkA = (ridxA, cidxA, gidxA, isem0)
    blkB = (ridxB, cidxB, gidxB, isem1)

    def load_blk(B, q):
        ridx, cidx, gidx, isem = B

        @pl.when(4 * q < n)
        def _():
            r0 = start + 4 * q
            d1 = pltpu.async_copy(row_hbm.at[pl.ds(r0, 4), :], ridx, isem)
            d2 = pltpu.async_copy(col_hbm.at[pl.ds(r0, 4), :], cidx, isem)
            d1.wait()
            d2.wait()
            for m in range(4):
                for j in range(CHUNK // 16):
                    sl = pl.ds(j * 16, 16)
                    ridx[m, sl] = ridx[m, sl] + cbase
                    gidx[m, sl] = cidx[m, sl] + cbase

    def prefetch(k, S, B, m):
        gx, gy, eb, h, gsem, esem, ssem = S
        ridx, cidx, gidx, isem = B

        @pl.when(k < n)
        def _():
            off = (start + k) * CHUNK
            pltpu.async_copy(ec_hbm.at[pl.ds(off, CHUNK), pl.ds(c * HH, HH)],
                             eb, esem)
            pltpu.async_copy(xas_hbm.at[ridx.at[m]], gx, gsem)
            pltpu.async_copy(xbs_hbm.at[gidx.at[m]], gy, gsem)

    def process(k, S, B, m):
        gx, gy, eb, h, gsem, esem, ssem = S
        ridx, cidx, gidx, isem = B

        @pl.when(k < n)
        def _():
            pltpu.make_async_copy(xas_hbm.at[ridx.at[m]], gx, gsem).wait()
            pltpu.make_async_copy(xbs_hbm.at[gidx.at[m]], gy, gsem).wait()
            pltpu.make_async_copy(ec_hbm.at[pl.ds(0, CHUNK), pl.ds(0, HH)],
                                  eb, esem).wait()

            @pl.when(k >= 2)
            def _():
                pltpu.make_async_copy(h, acc.at[cidx.at[m]], ssem).wait()

            @plsc.parallel_loop(0, CHUNK, unroll=2)
            def row_body(i):
                for j in range(HH // 16):
                    sl = pl.ds(j * 16, 16)
                    h[i, sl] = jnp.maximum(gx[i, sl] + gy[i, sl] + eb[i, sl],
                                           0.0)
            pltpu.async_copy(h, acc.at[cidx.at[m]], ssem, add=True)

    load_blk(blkA, jnp.int32(0))
    prefetch(jnp.int32(0), sets[0], blkA, 0)

    def oct_body(J, carry):
        k0 = 8 * J
        for m in range(3):
            prefetch(k0 + m + 1, sets[(m + 1) % 2], blkA, m + 1)
            process(k0 + m, sets[m % 2], blkA, m)
        load_blk(blkB, 2 * J + 1)
        prefetch(k0 + 4, sets[0], blkB, 0)
        process(k0 + 3, sets[1], blkA, 3)
        for m in range(3):
            prefetch(k0 + m + 5, sets[(m + 1) % 2], blkB, m + 1)
            process(k0 + m + 4, sets[m % 2], blkB, m)
        load_blk(blkA, 2 * J + 2)
        prefetch(k0 + 8, sets[0], blkA, 0)
        process(k0 + 7, sets[1], blkB, 3)
        return carry

    lax.fori_loop(0, (n + 7) // 8, oct_body, 0)

    pltpu.make_async_copy(h0, acc.at[cidxA.at[0]], ssem0).wait()
    pltpu.make_async_copy(h1, acc.at[cidxA.at[1]], ssem1).wait()
    plsc.subcore_barrier()

    def export_body(t, carry):
        r0 = s * ROWS_PER_TILE + t * CHUNK
        pltpu.sync_copy(acc.at[pl.ds(r0, CHUNK), pl.ds(0, HH)], gx0)
        pltpu.sync_copy(gx0, msg_hbm.at[pl.ds(r0, CHUNK), pl.ds(c * HH, HH)])
        return carry

    lax.fori_loop(0, ROWS_PER_TILE // CHUNK, export_body, 0)
    pltpu.sync_copy(acc.at[pl.ds(s * ROWS_PER_TILE, ROWS_PER_TILE),
                           pl.ds(HH, 1)], cntb)
    pltpu.sync_copy(cntb, cnt_hbm.at[c, pl.ds(s * ROWS_PER_TILE,
                                              ROWS_PER_TILE), :])


def _sc_aggregate(xa, xb, row, col, ec, zeros):
    mesh = plsc.VectorSubcoreMesh(core_axis_name="c", subcore_axis_name="s")
    buf = [
        pltpu.VMEM((CHUNK, HH), jnp.float32),
        pltpu.VMEM((CHUNK, HH), jnp.float32),
        pltpu.VMEM((CHUNK, HH), jnp.float32),
        pltpu.VMEM((CHUNK, ACC_W), jnp.float32),
    ]
    extra = [pltpu.VMEM((4, CHUNK), jnp.int32)] * 6
    run = pl.kernel(
        _sc_body,
        mesh=mesh,
        compiler_params=pltpu.CompilerParams(use_tc_tiling_on_sc=False),
        out_type=[
            jax.ShapeDtypeStruct((NC * N_NODES, HH), jnp.float32),
            jax.ShapeDtypeStruct((NC * N_NODES, HH), jnp.float32),
            jax.ShapeDtypeStruct((N_PAD, H), jnp.float32),
            jax.ShapeDtypeStruct((NC, N_PAD, 1), jnp.float32),
        ],
        scratch_types=buf + buf + extra + [
            pltpu.VMEM_SHARED((N_PAD, ACC_W), jnp.float32),
            pltpu.VMEM((ROWS_PER_TILE, 1), jnp.float32),
    ] + [pltpu.SemaphoreType.DMA] * 8,
    )
    out = run(xa, xb, row, col, ec, zeros)
    return out[2], out[3]


def kernel(x, edge_index, edge_attr, W1, b1, W2, b2):
    w1a = W1[:H]
    w1b = W1[H:2 * H]
    w1c = W1[2 * H:]
    w2a = W2[:H]
    w2b = W2[H:]
    b1r = b1.reshape(1, H)
    b2r = b2.reshape(1, H)

    xa, xb = _node_tables(x, w1a, w1b)
    ec = _edge_table(edge_attr, w1c, b1r)
    zeros = jnp.zeros((ROWS_PER_TILE, ACC_W), jnp.float32)
    row2 = edge_index[0].reshape(N_CHUNKS, CHUNK)
    col2 = edge_index[1].reshape(N_CHUNKS, CHUNK)
    msg, cnt = _sc_aggregate(xa, xb, row2, col2, ec, zeros)
    return _finalize(x, msg, cnt, w2a, w2b, b2r)
